# Initial kernel scaffold; baseline (speedup 1.0000x reference)
import jax
import jax.numpy as jnp
from jax.experimental import pallas as pl


def kernel(features, W, b):
    raise NotImplementedError("write your pallas kernel here")


# speedup vs baseline: 2.0152x; 2.0152x over previous
import jax
import jax.numpy as jnp
from jax import lax
from jax.experimental import pallas as pl

N = 100000
D = 256
B = 4
TOPK = 5
BN = 2000
NBLK = N // BN


def _logits_body(f_ref, w_ref, b_ref, out_ref):
    lt = lax.dot_general(
        w_ref[...], f_ref[...],
        dimension_numbers=(((1,), (1,)), ((), ())),
        preferred_element_type=jnp.float32,
    ) + b_ref[...]
    out_ref[...] = lt.reshape(1, B, BN)


def _mask_body(l_ref, out_ref):
    l = l_ref[...]
    m = jnp.max(jnp.max(l, axis=2, keepdims=True), axis=0, keepdims=True)
    e = jnp.exp(l - m)
    s = jnp.sum(jnp.sum(e, axis=2, keepdims=True), axis=0, keepdims=True)
    w = jnp.mean(e / s, axis=1, keepdims=True)
    for _ in range(TOPK):
        mx = jnp.max(w)
        w = jnp.where(w == mx, 0.0, w)
    m2 = jnp.max(w)
    e2 = jnp.exp(w - m2)
    out_ref[...] = e2 / jnp.sum(e2)


def _pool_body(w_ref, f_ref, out_ref):
    @pl.when(pl.program_id(0) == 0)
    def _():
        out_ref[...] = jnp.zeros_like(out_ref)

    out_ref[...] += lax.dot_general(
        w_ref[...].reshape(1, BN), f_ref[...],
        dimension_numbers=(((1,), (0,)), ((), ())),
        preferred_element_type=jnp.float32,
    )


def kernel(features, W, b):
    logits = pl.pallas_call(
        _logits_body,
        grid=(NBLK,),
        in_specs=[
            pl.BlockSpec((BN, D), lambda i: (i, 0)),
            pl.BlockSpec((B, D), lambda i: (0, 0)),
            pl.BlockSpec((B, 1), lambda i: (0, 0)),
        ],
        out_specs=pl.BlockSpec((1, B, BN), lambda i: (i, 0, 0)),
        out_shape=jax.ShapeDtypeStruct((NBLK, B, BN), jnp.float32),
    )(features, W, b.reshape(B, 1))

    w2 = pl.pallas_call(
        _mask_body,
        in_specs=[pl.BlockSpec((NBLK, B, BN), lambda: (0, 0, 0))],
        out_specs=pl.BlockSpec((NBLK, 1, BN), lambda: (0, 0, 0)),
        out_shape=jax.ShapeDtypeStruct((NBLK, 1, BN), jnp.float32),
    )(logits)

    bag = pl.pallas_call(
        _pool_body,
        grid=(NBLK,),
        in_specs=[
            pl.BlockSpec((1, 1, BN), lambda i: (i, 0, 0)),
            pl.BlockSpec((BN, D), lambda i: (i, 0)),
        ],
        out_specs=pl.BlockSpec((1, D), lambda i: (0, 0)),
        out_shape=jax.ShapeDtypeStruct((1, D), jnp.float32),
    )(w2, features)

    return (bag.reshape(D), w2.reshape(N))


# speedup vs baseline: 2.0761x; 1.0302x over previous
import jax
import jax.numpy as jnp
from jax import lax
from jax.experimental import pallas as pl
from jax.experimental.pallas import tpu as pltpu

N = 100000
D = 256
B = 4
TOPK = 5
BN = 2000
NBLK = N // BN


def _body(f_ref, w_ref, b_ref, w2_ref, bag_ref, l_sc, w2_sc):
    i = pl.program_id(0)

    @pl.when(i < NBLK)
    def _logits_phase():
        lt = lax.dot_general(
            w_ref[...], f_ref[...],
            dimension_numbers=(((1,), (1,)), ((), ())),
            preferred_element_type=jnp.float32,
        ) + b_ref[...]
        l_sc[pl.ds(i, 1)] = lt.reshape(1, B, BN)

    @pl.when(i == NBLK - 1)
    def _mask_phase():
        l = l_sc[...]
        m = jnp.max(jnp.max(l, axis=2, keepdims=True), axis=0, keepdims=True)
        e = jnp.exp(l - m)
        s = jnp.sum(jnp.sum(e, axis=2, keepdims=True), axis=0, keepdims=True)
        w = jnp.mean(e / s, axis=1, keepdims=True)
        for _ in range(TOPK):
            mx = jnp.max(w)
            w = jnp.where(w == mx, 0.0, w)
        m2 = jnp.max(w)
        e2 = jnp.exp(w - m2)
        w2_sc[...] = e2 / jnp.sum(e2)

    @pl.when(i >= NBLK)
    def _pool_phase():
        @pl.when(i == NBLK)
        def _():
            bag_ref[...] = jnp.zeros_like(bag_ref)

        w2_blk = w2_sc[pl.ds(i - NBLK, 1)]
        w2_ref[...] = w2_blk
        bag_ref[...] += lax.dot_general(
            w2_blk.reshape(1, BN), f_ref[...],
            dimension_numbers=(((1,), (0,)), ((), ())),
            preferred_element_type=jnp.float32,
        )


def kernel(features, W, b):
    w2, bag = pl.pallas_call(
        _body,
        grid=(2 * NBLK,),
        in_specs=[
            pl.BlockSpec((BN, D), lambda i: (lax.rem(i, NBLK), 0)),
            pl.BlockSpec((B, D), lambda i: (0, 0)),
            pl.BlockSpec((B, 1), lambda i: (0, 0)),
        ],
        out_specs=[
            pl.BlockSpec((1, 1, BN), lambda i: (jnp.maximum(i - NBLK, 0), 0, 0)),
            pl.BlockSpec((1, D), lambda i: (0, 0)),
        ],
        out_shape=[
            jax.ShapeDtypeStruct((NBLK, 1, BN), jnp.float32),
            jax.ShapeDtypeStruct((1, D), jnp.float32),
        ],
        scratch_shapes=[
            pltpu.VMEM((NBLK, B, BN), jnp.float32),
            pltpu.VMEM((NBLK, 1, BN), jnp.float32),
        ],
    )(features, W, b.reshape(B, 1))

    return (bag.reshape(D), w2.reshape(N))


# speedup vs baseline: 2.9735x; 1.4323x over previous
import jax
import jax.numpy as jnp
from jax import lax
from jax.experimental import pallas as pl
from jax.experimental.pallas import tpu as pltpu

N = 100000
D = 256
B = 4
TOPK = 5
BN = 5000
NBLK = N // BN


def _body(f_ref, w_ref, b_ref, w2_ref, bag_ref, l_sc, w2_sc):
    i = pl.program_id(0)

    @pl.when(i < NBLK)
    def _logits_phase():
        lt = lax.dot_general(
            w_ref[...], f_ref[...],
            dimension_numbers=(((1,), (1,)), ((), ())),
            preferred_element_type=jnp.float32,
        ) + b_ref[...]
        l_sc[pl.ds(i, 1)] = lt.reshape(1, B, BN)

    @pl.when(i == NBLK - 1)
    def _mask_phase():
        l = l_sc[...]
        m = jnp.max(jnp.max(l, axis=2, keepdims=True), axis=0, keepdims=True)
        e = jnp.exp(l - m)
        s = jnp.sum(jnp.sum(e, axis=2, keepdims=True), axis=0, keepdims=True)
        w = jnp.mean(e / s, axis=1, keepdims=True)
        for _ in range(TOPK):
            mx = jnp.max(w)
            w = jnp.where(w == mx, 0.0, w)
        m2 = jnp.max(w)
        e2 = jnp.exp(w - m2)
        w2_sc[...] = e2 / jnp.sum(e2)

    @pl.when(i >= NBLK)
    def _pool_phase():
        @pl.when(i == NBLK)
        def _():
            bag_ref[...] = jnp.zeros_like(bag_ref)

        w2_blk = w2_sc[pl.ds(i - NBLK, 1)]
        w2_ref[...] = w2_blk
        bag_ref[...] += lax.dot_general(
            w2_blk.reshape(1, BN), f_ref[...],
            dimension_numbers=(((1,), (0,)), ((), ())),
            preferred_element_type=jnp.float32,
        )


def kernel(features, W, b):
    w2, bag = pl.pallas_call(
        _body,
        grid=(2 * NBLK,),
        in_specs=[
            pl.BlockSpec((BN, D), lambda i: (lax.rem(i, NBLK), 0)),
            pl.BlockSpec((B, D), lambda i: (0, 0)),
            pl.BlockSpec((B, 1), lambda i: (0, 0)),
        ],
        out_specs=[
            pl.BlockSpec((1, 1, BN), lambda i: (jnp.maximum(i - NBLK, 0), 0, 0)),
            pl.BlockSpec((1, D), lambda i: (0, 0)),
        ],
        out_shape=[
            jax.ShapeDtypeStruct((NBLK, 1, BN), jnp.float32),
            jax.ShapeDtypeStruct((1, D), jnp.float32),
        ],
        scratch_shapes=[
            pltpu.VMEM((NBLK, B, BN), jnp.float32),
            pltpu.VMEM((NBLK, 1, BN), jnp.float32),
        ],
    )(features, W, b.reshape(B, 1))

    return (bag.reshape(D), w2.reshape(N))


# speedup vs baseline: 3.2029x; 1.0771x over previous
import jax
import jax.numpy as jnp
from jax import lax
from jax.experimental import pallas as pl
from jax.experimental.pallas import tpu as pltpu

N = 100000
D = 256
B = 4
TOPK = 5
BN = 10000
NBLK = N // BN


def _body(f_ref, w_ref, b_ref, w2_ref, bag_ref, l_sc, w2_sc):
    i = pl.program_id(0)

    @pl.when(i < NBLK)
    def _logits_phase():
        lt = lax.dot_general(
            w_ref[...], f_ref[...],
            dimension_numbers=(((1,), (1,)), ((), ())),
            preferred_element_type=jnp.float32,
        ) + b_ref[...]
        l_sc[pl.ds(i, 1)] = lt.reshape(1, B, BN)

    @pl.when(i == NBLK - 1)
    def _mask_phase():
        l = l_sc[...]
        m = jnp.max(jnp.max(l, axis=2, keepdims=True), axis=0, keepdims=True)
        e = jnp.exp(l - m)
        s = jnp.sum(jnp.sum(e, axis=2, keepdims=True), axis=0, keepdims=True)
        w = jnp.mean(e / s, axis=1, keepdims=True)
        for _ in range(TOPK):
            mx = jnp.max(w)
            w = jnp.where(w == mx, 0.0, w)
        m2 = jnp.max(w)
        e2 = jnp.exp(w - m2)
        w2_sc[...] = e2 / jnp.sum(e2)

    @pl.when(i >= NBLK)
    def _pool_phase():
        @pl.when(i == NBLK)
        def _():
            bag_ref[...] = jnp.zeros_like(bag_ref)

        w2_blk = w2_sc[pl.ds(i - NBLK, 1)]
        w2_ref[...] = w2_blk
        bag_ref[...] += lax.dot_general(
            w2_blk.reshape(1, BN), f_ref[...],
            dimension_numbers=(((1,), (0,)), ((), ())),
            preferred_element_type=jnp.float32,
        )


def kernel(features, W, b):
    w2, bag = pl.pallas_call(
        _body,
        grid=(2 * NBLK,),
        in_specs=[
            pl.BlockSpec((BN, D), lambda i: (lax.rem(i, NBLK), 0)),
            pl.BlockSpec((B, D), lambda i: (0, 0)),
            pl.BlockSpec((B, 1), lambda i: (0, 0)),
        ],
        out_specs=[
            pl.BlockSpec((1, 1, BN), lambda i: (jnp.maximum(i - NBLK, 0), 0, 0)),
            pl.BlockSpec((1, D), lambda i: (0, 0)),
        ],
        out_shape=[
            jax.ShapeDtypeStruct((NBLK, 1, BN), jnp.float32),
            jax.ShapeDtypeStruct((1, D), jnp.float32),
        ],
        scratch_shapes=[
            pltpu.VMEM((NBLK, B, BN), jnp.float32),
            pltpu.VMEM((NBLK, 1, BN), jnp.float32),
        ],
    )(features, W, b.reshape(B, 1))

    return (bag.reshape(D), w2.reshape(N))
